# baseline (speedup 1.0000x reference)
import functools

import jax
import jax.numpy as jnp
from jax import lax
from jax.experimental import pallas as pl
from jax.experimental.pallas import tpu as pltpu

_G = 10
_SV = 1
_SE = 1


def _body(*refs):
    ctx_ref = refs[0]
    v_refs = refs[1:1 + _SV]
    e_refs = refs[1 + _SV:1 + _SV + _SE]
    w_ref, b_ref, o_ref, vacc, eacc = refs[1 + _SV + _SE:]
    i = pl.program_id(0)

    @pl.when(i == 0)
    def _init():
        vacc[...] = jnp.zeros_like(vacc)
        eacc[...] = jnp.zeros_like(eacc)

    s = jnp.sum(v_refs[0][...], axis=0, keepdims=True)
    for vr in v_refs[1:]:
        s += jnp.sum(vr[...], axis=0, keepdims=True)
    vacc[...] += s

    d_edge = e_refs[0].shape[0]
    ec = e_refs[0].shape[1]
    t = e_refs[0][...].reshape(d_edge, ec // 128, 128).sum(axis=1)
    for er in e_refs[1:]:
        t += er[...].reshape(d_edge, ec // 128, 128).sum(axis=1)
    eacc[...] += t

    @pl.when(i == _G - 1)
    def _finish():
        d_ctx = ctx_ref.shape[1]
        d_feat = vacc.shape[1]
        n_v = v_refs[0].shape[0] * _SV * _G
        n_e = ec * _SE * _G
        out = jnp.dot(ctx_ref[...], w_ref[0:d_ctx],
                      preferred_element_type=jnp.float32)
        out += jnp.dot(vacc[...] / n_v, w_ref[d_ctx:d_ctx + d_feat],
                       preferred_element_type=jnp.float32)
        e_sum = jnp.sum(eacc[...], axis=1, keepdims=True) / n_e
        out += lax.dot_general(
            e_sum, w_ref[d_ctx + d_feat:d_ctx + d_feat + d_edge],
            dimension_numbers=(((0,), (0,)), ((), ())),
            preferred_element_type=jnp.float32)
        o_ref[...] = out + b_ref[...]


def kernel(context, vertex_data, edge_data, W, b):
    n_verts, d_feat = vertex_data.shape
    n_edges, d_edge = edge_data.shape
    d_ctx = context.shape[0]
    d_tot = W.shape[0]

    edge_t = edge_data.T
    vc = n_verts // (_SV * _G)
    ec = n_edges // (_SE * _G)

    def _vmap(j):
        return lambda i, j=j: (_G * j + i, 0)

    def _emap(j):
        return lambda i, j=j: (0, _G * j + i)

    out = pl.pallas_call(
        _body,
        grid=(_G,),
        in_specs=(
            [pl.BlockSpec((1, d_ctx), lambda i: (0, 0))]
            + [pl.BlockSpec((vc, d_feat), _vmap(j)) for j in range(_SV)]
            + [pl.BlockSpec((d_edge, ec), _emap(j)) for j in range(_SE)]
            + [pl.BlockSpec((d_tot, d_ctx), lambda i: (0, 0)),
               pl.BlockSpec((1, d_ctx), lambda i: (0, 0))]),
        out_specs=pl.BlockSpec((1, d_ctx), lambda i: (0, 0)),
        out_shape=jax.ShapeDtypeStruct((1, d_ctx), jnp.float32),
        scratch_shapes=[pltpu.VMEM((1, d_feat), jnp.float32),
                        pltpu.VMEM((d_edge, 128), jnp.float32)],
    )(context.reshape(1, d_ctx), *([vertex_data] * _SV),
      *([edge_t] * _SE), W, b.reshape(1, d_ctx))

    return out.reshape(d_ctx)


# speedup vs baseline: 1.1539x; 1.1539x over previous
import functools

import jax
import jax.numpy as jnp
from jax import lax
from jax.experimental import pallas as pl
from jax.experimental.pallas import tpu as pltpu

_G = 2
_SV = 5
_SE = 10


def _body(*refs):
    ctx_ref = refs[0]
    v_refs = refs[1:1 + _SV]
    e_refs = refs[1 + _SV:1 + _SV + _SE]
    w_ref, b_ref, o_ref, vacc, eacc = refs[1 + _SV + _SE:]
    i = pl.program_id(0)

    @pl.when(i == 0)
    def _init():
        vacc[...] = jnp.zeros_like(vacc)
        eacc[...] = jnp.zeros_like(eacc)

    s = jnp.sum(v_refs[0][...], axis=0, keepdims=True)
    for vr in v_refs[1:]:
        s += jnp.sum(vr[...], axis=0, keepdims=True)
    vacc[...] += s

    d_edge = e_refs[0].shape[0]
    ec = e_refs[0].shape[1]
    t = e_refs[0][...].reshape(d_edge, ec // 128, 128).sum(axis=1)
    for er in e_refs[1:]:
        t += er[...].reshape(d_edge, ec // 128, 128).sum(axis=1)
    eacc[...] += t

    @pl.when(i == _G - 1)
    def _finish():
        d_ctx = ctx_ref.shape[1]
        d_feat = vacc.shape[1]
        n_v = v_refs[0].shape[0] * _SV * _G
        n_e = ec * _SE * _G
        out = jnp.dot(ctx_ref[...], w_ref[0:d_ctx],
                      preferred_element_type=jnp.float32)
        out += jnp.dot(vacc[...] / n_v, w_ref[d_ctx:d_ctx + d_feat],
                       preferred_element_type=jnp.float32)
        e_sum = jnp.sum(eacc[...], axis=1, keepdims=True) / n_e
        out += lax.dot_general(
            e_sum, w_ref[d_ctx + d_feat:d_ctx + d_feat + d_edge],
            dimension_numbers=(((0,), (0,)), ((), ())),
            preferred_element_type=jnp.float32)
        o_ref[...] = out + b_ref[...]


def kernel(context, vertex_data, edge_data, W, b):
    n_verts, d_feat = vertex_data.shape
    n_edges, d_edge = edge_data.shape
    d_ctx = context.shape[0]
    d_tot = W.shape[0]

    edge_t = edge_data.T
    vc = n_verts // (_SV * _G)
    ec = n_edges // (_SE * _G)

    def _vmap(j):
        return lambda i, j=j: (_G * j + i, 0)

    def _emap(j):
        return lambda i, j=j: (0, _G * j + i)

    out = pl.pallas_call(
        _body,
        grid=(_G,),
        in_specs=(
            [pl.BlockSpec((1, d_ctx), lambda i: (0, 0))]
            + [pl.BlockSpec((vc, d_feat), _vmap(j)) for j in range(_SV)]
            + [pl.BlockSpec((d_edge, ec), _emap(j)) for j in range(_SE)]
            + [pl.BlockSpec((d_tot, d_ctx), lambda i: (0, 0)),
               pl.BlockSpec((1, d_ctx), lambda i: (0, 0))]),
        out_specs=pl.BlockSpec((1, d_ctx), lambda i: (0, 0)),
        out_shape=jax.ShapeDtypeStruct((1, d_ctx), jnp.float32),
        scratch_shapes=[pltpu.VMEM((1, d_feat), jnp.float32),
                        pltpu.VMEM((d_edge, 128), jnp.float32)],
    )(context.reshape(1, d_ctx), *([vertex_data] * _SV),
      *([edge_t] * _SE), W, b.reshape(1, d_ctx))

    return out.reshape(d_ctx)


# speedup vs baseline: 1.1659x; 1.0104x over previous
import functools

import jax
import jax.numpy as jnp
from jax import lax
from jax.experimental import pallas as pl
from jax.experimental.pallas import tpu as pltpu

_G = 2
_SV = 5
_SE = 5


def _body(*refs):
    ctx_ref = refs[0]
    v_refs = refs[1:1 + _SV]
    e_refs = refs[1 + _SV:1 + _SV + _SE]
    w_ref, b_ref, o_ref, vacc, eacc = refs[1 + _SV + _SE:]
    i = pl.program_id(0)

    @pl.when(i == 0)
    def _init():
        vacc[...] = jnp.zeros_like(vacc)
        eacc[...] = jnp.zeros_like(eacc)

    s = jnp.sum(v_refs[0][...], axis=0, keepdims=True)
    for vr in v_refs[1:]:
        s += jnp.sum(vr[...], axis=0, keepdims=True)
    vacc[...] += s

    d_edge = e_refs[0].shape[0]
    ec = e_refs[0].shape[1]
    t = e_refs[0][...].reshape(d_edge, ec // 128, 128).sum(axis=1)
    for er in e_refs[1:]:
        t += er[...].reshape(d_edge, ec // 128, 128).sum(axis=1)
    eacc[...] += t

    @pl.when(i == _G - 1)
    def _finish():
        d_ctx = ctx_ref.shape[1]
        d_feat = vacc.shape[1]
        n_v = v_refs[0].shape[0] * _SV * _G
        n_e = ec * _SE * _G
        out = jnp.dot(ctx_ref[...], w_ref[0:d_ctx],
                      preferred_element_type=jnp.float32)
        out += jnp.dot(vacc[...] / n_v, w_ref[d_ctx:d_ctx + d_feat],
                       preferred_element_type=jnp.float32)
        e_sum = jnp.sum(eacc[...], axis=1, keepdims=True) / n_e
        out += lax.dot_general(
            e_sum, w_ref[d_ctx + d_feat:d_ctx + d_feat + d_edge],
            dimension_numbers=(((0,), (0,)), ((), ())),
            preferred_element_type=jnp.float32)
        o_ref[...] = out + b_ref[...]


def kernel(context, vertex_data, edge_data, W, b):
    n_verts, d_feat = vertex_data.shape
    n_edges, d_edge = edge_data.shape
    d_ctx = context.shape[0]
    d_tot = W.shape[0]

    edge_t = edge_data.T
    vc = n_verts // (_SV * _G)
    ec = n_edges // (_SE * _G)

    def _vmap(j):
        return lambda i, j=j: (_G * j + i, 0)

    def _emap(j):
        return lambda i, j=j: (0, _G * j + i)

    out = pl.pallas_call(
        _body,
        grid=(_G,),
        in_specs=(
            [pl.BlockSpec((1, d_ctx), lambda i: (0, 0))]
            + [pl.BlockSpec((vc, d_feat), _vmap(j)) for j in range(_SV)]
            + [pl.BlockSpec((d_edge, ec), _emap(j)) for j in range(_SE)]
            + [pl.BlockSpec((d_tot, d_ctx), lambda i: (0, 0)),
               pl.BlockSpec((1, d_ctx), lambda i: (0, 0))]),
        out_specs=pl.BlockSpec((1, d_ctx), lambda i: (0, 0)),
        out_shape=jax.ShapeDtypeStruct((1, d_ctx), jnp.float32),
        scratch_shapes=[pltpu.VMEM((1, d_feat), jnp.float32),
                        pltpu.VMEM((d_edge, 128), jnp.float32)],
    )(context.reshape(1, d_ctx), *([vertex_data] * _SV),
      *([edge_t] * _SE), W, b.reshape(1, d_ctx))

    return out.reshape(d_ctx)
